# baseline (speedup 1.0000x reference)
import functools

import jax
import jax.numpy as jnp
from jax import lax
from jax.experimental import pallas as pl
from jax.experimental.pallas import tpu as pltpu
from jax.experimental.pallas import tpu_sc as plsc

NUM_EMBEDDINGS = 1000000
DIM = 32
PADW = 128
BATCH = 16384
FIELDS = 100
NC = 2
NS = 16
NW = NC * NS
B_PER_W = BATCH // NW
IDXBLK = 64
NIDX = B_PER_W // IDXBLK
NBUF = 4

_mesh = plsc.VectorSubcoreMesh(core_axis_name="c", subcore_axis_name="s")


@functools.partial(
    pl.kernel,
    mesh=_mesh,
    out_type=jax.ShapeDtypeStruct((BATCH, DIM, FIELDS), jnp.float32),
    compiler_params=pltpu.CompilerParams(
        use_tc_tiling_on_sc=True, needs_layout_passes=False
    ),
    scratch_types=[
        pltpu.VMEM((2, IDXBLK, FIELDS), jnp.int32),
        pltpu.VMEM((NBUF, FIELDS, PADW), jnp.float32),
        pltpu.VMEM((NBUF, DIM, FIELDS), jnp.float32),
        pltpu.SemaphoreType.DMA,
        pltpu.SemaphoreType.DMA,
        pltpu.SemaphoreType.DMA,
    ],
)
def _emb_lookup(idx_hbm, table_hbm, out_hbm, idx_v, rows_v, cpt_v, isem,
                gsem, osem):
    wid = lax.axis_index("s") * NC + lax.axis_index("c")
    base = wid * B_PER_W

    def start_idx(g, slot):
        pltpu.async_copy(
            idx_hbm.at[pl.ds(base + g * IDXBLK, IDXBLK)], idx_v.at[slot], isem
        )

    def wait_idx(g, slot):
        pltpu.make_async_copy(
            idx_hbm.at[pl.ds(base + g * IDXBLK, IDXBLK)], idx_v.at[slot], isem
        ).wait()

    def start_gather(g_slot, bb, slot):
        pltpu.async_copy(
            table_hbm.at[idx_v.at[g_slot, bb]], rows_v.at[slot], gsem
        )

    def wait_gather(g_slot, bb, slot):
        pltpu.make_async_copy(
            table_hbm.at[idx_v.at[g_slot, bb]], rows_v.at[slot], gsem
        ).wait()

    _FSTARTS = (0, 16, 32, 48, 64, 80, 84)
    _iota = lax.iota(jnp.int32, 16)

    def compact(slot):
        def dim_col(d, carry):
            idx_d = jnp.full((16,), 0, jnp.int32) + d
            for f0 in _FSTARTS:
                val = plsc.load_gather(
                    rows_v.at[slot], [_iota + f0, idx_d]
                )
                cpt_v[slot, d, pl.ds(f0, 16)] = val
            return carry

        lax.fori_loop(0, DIM, dim_col, 0)

    def start_store(b, slot):
        pltpu.async_copy(cpt_v.at[slot], out_hbm.at[b], osem)

    def wait_store(b, slot):
        pltpu.make_async_copy(cpt_v.at[slot], out_hbm.at[b], osem).wait()

    start_idx(0, 0)

    def idx_block(g, carry):
        g_slot = g % 2
        wait_idx(g, g_slot)

        @pl.when(g + 1 < NIDX)
        def _():
            start_idx(g + 1, (g + 1) % 2)

        def slot_free(bb):
            bprev = g * IDXBLK + bb - NBUF

            @pl.when(bprev >= 0)
            def _():
                wait_store(base + bprev, bprev % NBUF)

        for p in range(2):
            slot_free(p)
            start_gather(g_slot, p, p % NBUF)

        def batch(bb, carry):
            b = base + g * IDXBLK + bb
            slot = bb % NBUF

            @pl.when(bb + 2 < IDXBLK)
            def _():
                slot_free(bb + 2)
                start_gather(g_slot, bb + 2, (bb + 2) % NBUF)

            wait_gather(g_slot, bb, slot)
            compact(slot)
            start_store(b, slot)
            return carry

        lax.fori_loop(0, IDXBLK, batch, 0)
        return carry

    lax.fori_loop(0, NIDX, idx_block, 0, unroll=2)

    def drain(k, carry):
        b = base + B_PER_W - NBUF + k
        wait_store(b, (B_PER_W - NBUF + k) % NBUF)
        return carry

    lax.fori_loop(0, NBUF, drain, 0)


def kernel(input_, weight):
    wpad = jax.lax.pad(
        weight, jnp.float32(0), ((0, 0, 0), (0, PADW - DIM, 0))
    )
    return jnp.transpose(_emb_lookup(input_.astype(jnp.int32), wpad),
                         (0, 2, 1))


# speedup vs baseline: 1.3169x; 1.3169x over previous
import functools

import jax
import jax.numpy as jnp
from jax import lax
from jax.experimental import pallas as pl
from jax.experimental.pallas import tpu as pltpu
from jax.experimental.pallas import tpu_sc as plsc

NUM_EMBEDDINGS = 1000000
DIM = 32
PADW = 128
BATCH = 16384
FIELDS = 100
NC = 2
NS = 16
NW = NC * NS
B_PER_W = BATCH // NW
IDXBLK = 64
NIDX = B_PER_W // IDXBLK
NBUF = 4

_mesh = plsc.VectorSubcoreMesh(core_axis_name="c", subcore_axis_name="s")


@functools.partial(
    pl.kernel,
    mesh=_mesh,
    out_type=jax.ShapeDtypeStruct((BATCH, FIELDS, DIM), jnp.float32),
    compiler_params=pltpu.CompilerParams(use_tc_tiling_on_sc=True),
    scratch_types=[
        pltpu.VMEM((2, IDXBLK, FIELDS), jnp.int32),
        pltpu.VMEM((NBUF, FIELDS, PADW), jnp.float32),
        pltpu.VMEM((NBUF, FIELDS, DIM), jnp.float32),
        pltpu.SemaphoreType.DMA,
        pltpu.SemaphoreType.DMA,
        pltpu.SemaphoreType.DMA,
    ],
)
def _emb_lookup(idx_hbm, table_hbm, out_hbm, idx_v, rows_v, cpt_v, isem,
                gsem, osem):
    wid = lax.axis_index("s") * NC + lax.axis_index("c")
    base = wid * B_PER_W

    def start_idx(g, slot):
        pltpu.async_copy(
            idx_hbm.at[pl.ds(base + g * IDXBLK, IDXBLK)], idx_v.at[slot], isem
        )

    def wait_idx(g, slot):
        pltpu.make_async_copy(
            idx_hbm.at[pl.ds(base + g * IDXBLK, IDXBLK)], idx_v.at[slot], isem
        ).wait()

    def start_gather(g_slot, bb, slot):
        pltpu.async_copy(
            table_hbm.at[idx_v.at[g_slot, bb]], rows_v.at[slot], gsem
        )

    def wait_gather(g_slot, bb, slot):
        pltpu.make_async_copy(
            table_hbm.at[idx_v.at[g_slot, bb]], rows_v.at[slot], gsem
        ).wait()

    def compact(slot):
        def row(i, carry):
            cpt_v[slot, i, pl.ds(0, 16)] = rows_v[slot, i, pl.ds(0, 16)]
            cpt_v[slot, i, pl.ds(16, 16)] = rows_v[slot, i, pl.ds(16, 16)]
            return carry

        lax.fori_loop(0, FIELDS, row, 0)

    def start_store(b, slot):
        pltpu.async_copy(cpt_v.at[slot], out_hbm.at[b], osem)

    def wait_store(b, slot):
        pltpu.make_async_copy(cpt_v.at[slot], out_hbm.at[b], osem).wait()

    start_idx(0, 0)

    def idx_block(g, carry):
        g_slot = g % 2
        wait_idx(g, g_slot)

        @pl.when(g + 1 < NIDX)
        def _():
            start_idx(g + 1, (g + 1) % 2)

        def slot_free(bb):
            bprev = g * IDXBLK + bb - NBUF

            @pl.when(bprev >= 0)
            def _():
                wait_store(base + bprev, bprev % NBUF)

        for p in range(2):
            slot_free(p)
            start_gather(g_slot, p, p % NBUF)

        def batch(bb, carry):
            b = base + g * IDXBLK + bb
            slot = bb % NBUF

            @pl.when(bb + 2 < IDXBLK)
            def _():
                slot_free(bb + 2)
                start_gather(g_slot, bb + 2, (bb + 2) % NBUF)

            wait_gather(g_slot, bb, slot)
            compact(slot)
            start_store(b, slot)
            return carry

        lax.fori_loop(0, IDXBLK, batch, 0)
        return carry

    lax.fori_loop(0, NIDX, idx_block, 0, unroll=2)

    def drain(k, carry):
        b = base + B_PER_W - NBUF + k
        wait_store(b, (B_PER_W - NBUF + k) % NBUF)
        return carry

    lax.fori_loop(0, NBUF, drain, 0)


TBK = 2048


def _padT_body(in_ref, out_ref):
    out_ref[:, :DIM] = jnp.transpose(in_ref[...], (1, 0))
    out_ref[:, DIM:] = jnp.zeros((TBK, PADW - DIM), jnp.float32)


def _pad_table(wT):
    return pl.pallas_call(
        _padT_body,
        grid=(pl.cdiv(NUM_EMBEDDINGS, TBK),),
        in_specs=[pl.BlockSpec((DIM, TBK), lambda i: (0, i))],
        out_specs=pl.BlockSpec((TBK, PADW), lambda i: (i, 0)),
        out_shape=jax.ShapeDtypeStruct((NUM_EMBEDDINGS, PADW), jnp.float32),
    )(wT)


def kernel(input_, weight):
    wpad = _pad_table(weight.T)
    return _emb_lookup(input_.astype(jnp.int32), wpad)


# speedup vs baseline: 1.8619x; 1.4139x over previous
import functools

import jax
import jax.numpy as jnp
from jax import lax
from jax.experimental import pallas as pl
from jax.experimental.pallas import tpu as pltpu
from jax.experimental.pallas import tpu_sc as plsc

NUM_EMBEDDINGS = 1000000
DIM = 32
PADW = 128
BATCH = 16384
FIELDS = 100
NC = 2
NS = 16
NW = NC * NS
B_PER_W = BATCH // NW
IDXBLK = 64
NIDX = B_PER_W // IDXBLK
NBUF = 4

_mesh = plsc.VectorSubcoreMesh(core_axis_name="c", subcore_axis_name="s")


@functools.partial(
    pl.kernel,
    mesh=_mesh,
    out_type=jax.ShapeDtypeStruct((BATCH, DIM, FIELDS), jnp.float32),
    compiler_params=pltpu.CompilerParams(
        use_tc_tiling_on_sc=True, needs_layout_passes=False
    ),
    scratch_types=[
        pltpu.VMEM((2, IDXBLK, FIELDS), jnp.int32),
        pltpu.VMEM((NBUF, FIELDS, PADW), jnp.float32),
        pltpu.VMEM((NBUF, DIM, FIELDS), jnp.float32),
        pltpu.SemaphoreType.DMA,
        pltpu.SemaphoreType.DMA,
        pltpu.SemaphoreType.DMA,
    ],
)
def _emb_lookup(idx_hbm, table_hbm, out_hbm, idx_v, rows_v, cpt_v, isem,
                gsem, osem):
    wid = lax.axis_index("s") * NC + lax.axis_index("c")
    base = wid * B_PER_W

    def start_idx(g, slot):
        pltpu.async_copy(
            idx_hbm.at[pl.ds(base + g * IDXBLK, IDXBLK)], idx_v.at[slot], isem
        )

    def wait_idx(g, slot):
        pltpu.make_async_copy(
            idx_hbm.at[pl.ds(base + g * IDXBLK, IDXBLK)], idx_v.at[slot], isem
        ).wait()

    def start_gather(g_slot, bb, slot):
        pltpu.async_copy(
            table_hbm.at[idx_v.at[g_slot, bb]], rows_v.at[slot], gsem
        )

    def wait_gather(g_slot, bb, slot):
        pltpu.make_async_copy(
            table_hbm.at[idx_v.at[g_slot, bb]], rows_v.at[slot], gsem
        ).wait()

    _FSTARTS = (0, 16, 32, 48, 64, 80, 84)
    _iota = lax.iota(jnp.int32, 16)

    def compact(slot):
        rows2d = rows_v.at[slot]
        cpt2d = cpt_v.at[slot]

        def diag(k, carry):
            colp = lax.bitwise_and(_iota + k, 15)
            for d0 in (0, 16):
                cidx = colp + d0
                for f0 in _FSTARTS:
                    fidx = _iota + f0
                    val = plsc.load_gather(rows2d, [fidx, cidx])
                    plsc.store_scatter(cpt2d, [cidx, fidx], val)
            return carry

        lax.fori_loop(0, 16, diag, 0)

    def start_store(b, slot):
        pltpu.async_copy(cpt_v.at[slot], out_hbm.at[b], osem)

    def wait_store(b, slot):
        pltpu.make_async_copy(cpt_v.at[slot], out_hbm.at[b], osem).wait()

    start_idx(0, 0)

    def idx_block(g, carry):
        g_slot = g % 2
        wait_idx(g, g_slot)

        @pl.when(g + 1 < NIDX)
        def _():
            start_idx(g + 1, (g + 1) % 2)

        def slot_free(bb):
            bprev = g * IDXBLK + bb - NBUF

            @pl.when(bprev >= 0)
            def _():
                wait_store(base + bprev, bprev % NBUF)

        for p in range(2):
            slot_free(p)
            start_gather(g_slot, p, p % NBUF)

        def batch(bb, carry):
            b = base + g * IDXBLK + bb
            slot = bb % NBUF

            @pl.when(bb + 2 < IDXBLK)
            def _():
                slot_free(bb + 2)
                start_gather(g_slot, bb + 2, (bb + 2) % NBUF)

            wait_gather(g_slot, bb, slot)
            compact(slot)
            start_store(b, slot)
            return carry

        lax.fori_loop(0, IDXBLK, batch, 0)
        return carry

    lax.fori_loop(0, NIDX, idx_block, 0, unroll=2)

    def drain(k, carry):
        b = base + B_PER_W - NBUF + k
        wait_store(b, (B_PER_W - NBUF + k) % NBUF)
        return carry

    lax.fori_loop(0, NBUF, drain, 0)


TBK = 2048


def _padT_body(in_ref, out_ref):
    out_ref[:, :DIM] = jnp.transpose(in_ref[...], (1, 0))
    out_ref[:, DIM:] = jnp.zeros((TBK, PADW - DIM), jnp.float32)


def _pad_table(wT):
    return pl.pallas_call(
        _padT_body,
        grid=(pl.cdiv(NUM_EMBEDDINGS, TBK),),
        in_specs=[pl.BlockSpec((DIM, TBK), lambda i: (0, i))],
        out_specs=pl.BlockSpec((TBK, PADW), lambda i: (i, 0)),
        out_shape=jax.ShapeDtypeStruct((NUM_EMBEDDINGS, PADW), jnp.float32),
    )(wT)


def kernel(input_, weight):
    wpad = _pad_table(weight.T)
    return jnp.transpose(_emb_lookup(input_.astype(jnp.int32), wpad),
                         (0, 2, 1))


# speedup vs baseline: 2.2642x; 1.2160x over previous
import functools

import jax
import jax.numpy as jnp
from jax import lax
from jax.experimental import pallas as pl
from jax.experimental.pallas import tpu as pltpu
from jax.experimental.pallas import tpu_sc as plsc

NUM_EMBEDDINGS = 1000000
DIM = 32
PADW = 128
BATCH = 16384
FIELDS = 100
NC = 2
NS = 16
NW = NC * NS
B_PER_W = BATCH // NW
IDXBLK = 64
NIDX = B_PER_W // IDXBLK
NBUF = 4

_mesh = plsc.VectorSubcoreMesh(core_axis_name="c", subcore_axis_name="s")


@functools.partial(
    pl.kernel,
    mesh=_mesh,
    out_type=jax.ShapeDtypeStruct((BATCH, DIM, FIELDS), jnp.float32),
    compiler_params=pltpu.CompilerParams(
        use_tc_tiling_on_sc=True, needs_layout_passes=False
    ),
    scratch_types=[
        pltpu.VMEM((2, IDXBLK, FIELDS), jnp.int32),
        pltpu.VMEM((NBUF, FIELDS, PADW), jnp.float32),
        pltpu.VMEM((NBUF, DIM, FIELDS), jnp.float32),
        pltpu.SemaphoreType.DMA,
        pltpu.SemaphoreType.DMA,
        pltpu.SemaphoreType.DMA,
    ],
)
def _emb_lookup(idx_hbm, table_hbm, out_hbm, idx_v, rows_v, cpt_v, isem,
                gsem, osem):
    wid = lax.axis_index("s") * NC + lax.axis_index("c")
    base = wid * B_PER_W

    def start_idx(g, slot):
        pltpu.async_copy(
            idx_hbm.at[pl.ds(base + g * IDXBLK, IDXBLK)], idx_v.at[slot], isem
        )

    def wait_idx(g, slot):
        pltpu.make_async_copy(
            idx_hbm.at[pl.ds(base + g * IDXBLK, IDXBLK)], idx_v.at[slot], isem
        ).wait()

    def start_gather(g_slot, bb, slot):
        pltpu.async_copy(
            table_hbm.at[idx_v.at[g_slot, bb]], rows_v.at[slot], gsem
        )

    def wait_gather(g_slot, bb, slot):
        pltpu.make_async_copy(
            table_hbm.at[idx_v.at[g_slot, bb]], rows_v.at[slot], gsem
        ).wait()

    _FSTARTS = (0, 16, 32, 48, 64, 80, 84)
    _iota = lax.iota(jnp.int32, 16)

    def compact(slot):
        rows2d = rows_v.at[slot]
        cpt2d = cpt_v.at[slot]

        def diag(k, carry):
            colp = lax.bitwise_and(_iota + k, 15)
            for d0 in (0, 16):
                cidx = colp + d0
                for f0 in _FSTARTS:
                    fidx = _iota + f0
                    val = plsc.load_gather(rows2d, [fidx, cidx])
                    plsc.store_scatter(cpt2d, [cidx, fidx], val)
            return carry

        lax.fori_loop(0, 16, diag, 0)

    def start_store(b, slot):
        pltpu.async_copy(cpt_v.at[slot], out_hbm.at[b], osem)

    def wait_store(b, slot):
        pltpu.make_async_copy(cpt_v.at[slot], out_hbm.at[b], osem).wait()

    start_idx(0, 0)

    def idx_block(g, carry):
        g_slot = g % 2
        wait_idx(g, g_slot)

        @pl.when(g + 1 < NIDX)
        def _():
            start_idx(g + 1, (g + 1) % 2)

        def slot_free(bb):
            bprev = g * IDXBLK + bb - NBUF

            @pl.when(bprev >= 0)
            def _():
                wait_store(base + bprev, bprev % NBUF)

        for p in range(2):
            slot_free(p)
            start_gather(g_slot, p, p % NBUF)

        def batch(bb, carry):
            b = base + g * IDXBLK + bb
            slot = bb % NBUF

            @pl.when(bb + 2 < IDXBLK)
            def _():
                slot_free(bb + 2)
                start_gather(g_slot, bb + 2, (bb + 2) % NBUF)

            wait_gather(g_slot, bb, slot)
            compact(slot)
            start_store(b, slot)
            return carry

        lax.fori_loop(0, IDXBLK, batch, 0)
        return carry

    lax.fori_loop(0, NIDX, idx_block, 0, unroll=2)

    def drain(k, carry):
        b = base + B_PER_W - NBUF + k
        wait_store(b, (B_PER_W - NBUF + k) % NBUF)
        return carry

    lax.fori_loop(0, NBUF, drain, 0)


TBK = 8192


def _padT_body(in_ref, out_ref):
    out_ref[:, :DIM] = jnp.transpose(in_ref[...], (1, 0))


def _pad_table(wT):
    return pl.pallas_call(
        _padT_body,
        grid=(pl.cdiv(NUM_EMBEDDINGS, TBK),),
        in_specs=[pl.BlockSpec((DIM, TBK), lambda i: (0, i))],
        out_specs=pl.BlockSpec((TBK, PADW), lambda i: (i, 0)),
        out_shape=jax.ShapeDtypeStruct((NUM_EMBEDDINGS, PADW), jnp.float32),
    )(wT)


def kernel(input_, weight):
    wpad = _pad_table(weight.T)
    return jnp.transpose(_emb_lookup(input_.astype(jnp.int32), wpad),
                         (0, 2, 1))


# speedup vs baseline: 2.6546x; 1.1724x over previous
import functools

import jax
import jax.numpy as jnp
from jax import lax
from jax.experimental import pallas as pl
from jax.experimental.pallas import tpu as pltpu
from jax.experimental.pallas import tpu_sc as plsc

NUM_EMBEDDINGS = 1000000
DIM = 32
PADW = 128
BATCH = 16384
FIELDS = 100
NC = 2
NS = 16
NW = NC * NS
B_PER_W = BATCH // NW
IDXBLK = 128
NIDX = B_PER_W // IDXBLK
NBUF = 3

_FSTARTS = (0, 16, 32, 48, 64, 80, 84)

_mesh = plsc.VectorSubcoreMesh(core_axis_name="c", subcore_axis_name="s")


@functools.partial(
    pl.kernel,
    mesh=_mesh,
    out_type=jax.ShapeDtypeStruct((FIELDS, DIM, BATCH), jnp.float32),
    compiler_params=pltpu.CompilerParams(
        use_tc_tiling_on_sc=True, needs_layout_passes=False
    ),
    scratch_types=[
        pltpu.VMEM((2, IDXBLK, FIELDS), jnp.int32),
        pltpu.VMEM((FIELDS, IDXBLK), jnp.int32),
        pltpu.VMEM((NBUF, IDXBLK, PADW), jnp.float32),
        pltpu.VMEM((NBUF, DIM, IDXBLK), jnp.float32),
        pltpu.SemaphoreType.DMA,
        pltpu.SemaphoreType.DMA,
        pltpu.SemaphoreType.DMA,
    ],
)
def _emb_lookup(idx_hbm, table_hbm, out_hbm, idx_v, idxt_v, rows_v, cpt_v,
                isem, gsem, osem):
    wid = lax.axis_index("s") * NC + lax.axis_index("c")
    base = wid * B_PER_W
    iota = lax.iota(jnp.int32, 16)

    def diag_transpose(src, dst, rstarts, cstarts):
        def step(k, carry):
            colp = lax.bitwise_and(iota + k, 15)
            for r0 in rstarts:
                ridx = iota + r0
                for c0 in cstarts:
                    cidx = colp + c0
                    val = plsc.load_gather(src, [ridx, cidx])
                    plsc.store_scatter(dst, [cidx, ridx], val)
            return carry

        lax.fori_loop(0, 16, step, 0)

    def start_idx(g, slot):
        pltpu.async_copy(
            idx_hbm.at[pl.ds(base + g * IDXBLK, IDXBLK)], idx_v.at[slot], isem
        )

    def wait_idx(g, slot):
        pltpu.make_async_copy(
            idx_hbm.at[pl.ds(base + g * IDXBLK, IDXBLK)], idx_v.at[slot], isem
        ).wait()

    def start_gather(f, slot):
        pltpu.async_copy(table_hbm.at[idxt_v.at[f]], rows_v.at[slot], gsem)

    def wait_gather(f, slot):
        pltpu.make_async_copy(
            table_hbm.at[idxt_v.at[f]], rows_v.at[slot], gsem
        ).wait()

    def start_store(g, f, slot):
        pltpu.async_copy(
            cpt_v.at[slot],
            out_hbm.at[f, slice(None), pl.ds(base + g * IDXBLK, IDXBLK)],
            osem,
        )

    def wait_store(g, f, slot):
        pltpu.make_async_copy(
            cpt_v.at[slot],
            out_hbm.at[f, slice(None), pl.ds(base + g * IDXBLK, IDXBLK)],
            osem,
        ).wait()

    _RSTARTS = tuple(range(0, IDXBLK, 16))

    start_idx(0, 0)

    def idx_block(g, carry):
        g_slot = g % 2
        wait_idx(g, g_slot)
        diag_transpose(idx_v.at[g_slot], idxt_v, _RSTARTS, _FSTARTS)

        @pl.when(g + 1 < NIDX)
        def _():
            start_idx(g + 1, (g + 1) % 2)

        def slot_free(f):
            fprev = f - NBUF

            @pl.when((g > 0) | (fprev >= 0))
            def _():
                gp = jnp.where(fprev >= 0, g, g - 1)
                fp = jnp.where(fprev >= 0, fprev, FIELDS + fprev)
                wait_store(gp, fp, fprev % NBUF)

        for p in range(2):
            slot_free(p)
            start_gather(p, p % NBUF)

        def field(f, carry):
            slot = f % NBUF

            @pl.when(f + 2 < FIELDS)
            def _():
                slot_free(f + 2)
                start_gather(f + 2, (f + 2) % NBUF)

            wait_gather(f, slot)
            diag_transpose(rows_v.at[slot], cpt_v.at[slot], _RSTARTS, (0, 16))
            start_store(g, f, slot)
            return carry

        lax.fori_loop(0, FIELDS, field, 0)
        return carry

    lax.fori_loop(0, NIDX, idx_block, 0)

    def drain(k, carry):
        f = FIELDS - NBUF + k
        wait_store(NIDX - 1, f, f % NBUF)
        return carry

    lax.fori_loop(0, NBUF, drain, 0)


TBK = 8192


def _padT_body(in_ref, out_ref):
    out_ref[:, :DIM] = jnp.transpose(in_ref[...], (1, 0))


def _pad_table(wT):
    return pl.pallas_call(
        _padT_body,
        grid=(pl.cdiv(NUM_EMBEDDINGS, TBK),),
        in_specs=[pl.BlockSpec((DIM, TBK), lambda i: (0, i))],
        out_specs=pl.BlockSpec((TBK, PADW), lambda i: (i, 0)),
        out_shape=jax.ShapeDtypeStruct((NUM_EMBEDDINGS, PADW), jnp.float32),
    )(wT)


def kernel(input_, weight):
    wpad = _pad_table(weight.T)
    out = _emb_lookup(input_.astype(jnp.int32), wpad)
    return jnp.transpose(out, (2, 0, 1))


# speedup vs baseline: 2.7513x; 1.0364x over previous
import functools

import jax
import jax.numpy as jnp
from jax import lax
from jax.experimental import pallas as pl
from jax.experimental.pallas import tpu as pltpu
from jax.experimental.pallas import tpu_sc as plsc

NUM_EMBEDDINGS = 1000000
DIM = 32
PADW = 128
BATCH = 16384
FIELDS = 100
NC = 2
NS = 16
NW = NC * NS
B_PER_W = BATCH // NW
IDXBLK = 128
NIDX = B_PER_W // IDXBLK
NBUF = 3

_FSTARTS = (0, 16, 32, 48, 64, 80, 84)

_mesh = plsc.VectorSubcoreMesh(core_axis_name="c", subcore_axis_name="s")


@functools.partial(
    pl.kernel,
    mesh=_mesh,
    out_type=jax.ShapeDtypeStruct((FIELDS, DIM, BATCH), jnp.float32),
    compiler_params=pltpu.CompilerParams(
        use_tc_tiling_on_sc=True, needs_layout_passes=False
    ),
    scratch_types=[
        pltpu.VMEM((2, FIELDS, IDXBLK), jnp.int32),
        pltpu.VMEM((NBUF, IDXBLK, PADW), jnp.float32),
        pltpu.VMEM((NBUF, DIM, IDXBLK), jnp.float32),
        pltpu.SemaphoreType.DMA,
        pltpu.SemaphoreType.DMA,
        pltpu.SemaphoreType.DMA,
    ],
)
def _emb_lookup(idx_hbm, table_hbm, out_hbm, idxt_v, rows_v, cpt_v,
                isem, gsem, osem):
    wid = lax.axis_index("s") * NC + lax.axis_index("c")
    base = wid * B_PER_W
    iota = lax.iota(jnp.int32, 16)

    def diag_transpose(src, dst, rstarts, cstarts):
        def step(k, carry):
            colp = lax.bitwise_and(iota + k, 15)
            for r0 in rstarts:
                ridx = iota + r0
                for c0 in cstarts:
                    cidx = colp + c0
                    val = plsc.load_gather(src, [ridx, cidx])
                    plsc.store_scatter(dst, [cidx, ridx], val)
            return carry

        lax.fori_loop(0, 16, step, 0)

    def start_idx(g, slot):
        pltpu.async_copy(
            idx_hbm.at[slice(None), pl.ds(base + g * IDXBLK, IDXBLK)],
            idxt_v.at[slot], isem,
        )

    def wait_idx(g, slot):
        pltpu.make_async_copy(
            idx_hbm.at[slice(None), pl.ds(base + g * IDXBLK, IDXBLK)],
            idxt_v.at[slot], isem,
        ).wait()

    def start_gather(g_slot, f, slot):
        pltpu.async_copy(
            table_hbm.at[idxt_v.at[g_slot, f]], rows_v.at[slot], gsem
        )

    def wait_gather(g_slot, f, slot):
        pltpu.make_async_copy(
            table_hbm.at[idxt_v.at[g_slot, f]], rows_v.at[slot], gsem
        ).wait()

    def start_store(g, f, slot):
        pltpu.async_copy(
            cpt_v.at[slot],
            out_hbm.at[f, slice(None), pl.ds(base + g * IDXBLK, IDXBLK)],
            osem,
        )

    def wait_store(g, f, slot):
        pltpu.make_async_copy(
            cpt_v.at[slot],
            out_hbm.at[f, slice(None), pl.ds(base + g * IDXBLK, IDXBLK)],
            osem,
        ).wait()

    _RSTARTS = tuple(range(0, IDXBLK, 16))

    start_idx(0, 0)

    def idx_block(g, carry):
        g_slot = g % 2
        wait_idx(g, g_slot)

        @pl.when(g + 1 < NIDX)
        def _():
            start_idx(g + 1, (g + 1) % 2)

        def slot_free(f):
            fprev = f - NBUF

            @pl.when((g > 0) | (fprev >= 0))
            def _():
                gp = jnp.where(fprev >= 0, g, g - 1)
                fp = jnp.where(fprev >= 0, fprev, FIELDS + fprev)
                wait_store(gp, fp, fprev % NBUF)

        for p in range(2):
            slot_free(p)
            start_gather(g_slot, p, p % NBUF)

        def field(f, carry):
            slot = f % NBUF

            @pl.when(f + 2 < FIELDS)
            def _():
                slot_free(f + 2)
                start_gather(g_slot, f + 2, (f + 2) % NBUF)

            wait_gather(g_slot, f, slot)
            diag_transpose(rows_v.at[slot], cpt_v.at[slot], _RSTARTS, (0, 16))
            start_store(g, f, slot)
            return carry

        lax.fori_loop(0, FIELDS, field, 0)
        return carry

    lax.fori_loop(0, NIDX, idx_block, 0)

    def drain(k, carry):
        f = FIELDS - NBUF + k
        wait_store(NIDX - 1, f, f % NBUF)
        return carry

    lax.fori_loop(0, NBUF, drain, 0)


TBK = 8192


def _padT_body(in_ref, out_ref):
    out_ref[:, :DIM] = jnp.transpose(in_ref[...], (1, 0))


def _pad_table(wT):
    return pl.pallas_call(
        _padT_body,
        grid=(pl.cdiv(NUM_EMBEDDINGS, TBK),),
        in_specs=[pl.BlockSpec((DIM, TBK), lambda i: (0, i))],
        out_specs=pl.BlockSpec((TBK, PADW), lambda i: (i, 0)),
        out_shape=jax.ShapeDtypeStruct((NUM_EMBEDDINGS, PADW), jnp.float32),
    )(wT)


def kernel(input_, weight):
    wpad = _pad_table(weight.T)
    out = _emb_lookup(input_.astype(jnp.int32).T, wpad)
    return jnp.transpose(out, (2, 0, 1))
